# baseline (speedup 1.0000x reference)
import jax
import jax.numpy as jnp
from jax.experimental import pallas as pl
from jax.experimental.pallas import tpu as pltpu


def kernel(tokens, W_pos):
    B = tokens.shape[0]
    S = tokens.shape[1]
    D = W_pos.shape[1]

    def body(w_hbm, out_hbm, sem):
        copies = [pltpu.async_copy(w_hbm, out_hbm.at[b], sem) for b in range(B)]
        for c in copies:
            c.wait()

    return pl.pallas_call(
        body,
        in_specs=[pl.BlockSpec(memory_space=pltpu.MemorySpace.HBM)],
        out_specs=pl.BlockSpec(memory_space=pltpu.MemorySpace.HBM),
        out_shape=jax.ShapeDtypeStruct((B, S, D), W_pos.dtype),
        scratch_shapes=[
            pltpu.SemaphoreType.DMA,
        ],
    )(W_pos[:S])


# speedup vs baseline: 31.5915x; 31.5915x over previous
import jax
import jax.numpy as jnp
from jax.experimental import pallas as pl
from jax.experimental.pallas import tpu as pltpu

_CHUNKS = 4


def _body(w_ref, out_ref):
    out_ref[...] = w_ref[...][None]


def kernel(tokens, W_pos):
    B = tokens.shape[0]
    S = tokens.shape[1]
    D = W_pos.shape[1]
    rc = S // _CHUNKS
    return pl.pallas_call(
        _body,
        grid=(_CHUNKS, B),
        in_specs=[pl.BlockSpec((rc, D), lambda i, b: (i, 0))],
        out_specs=pl.BlockSpec((1, rc, D), lambda i, b: (b, i, 0)),
        out_shape=jax.ShapeDtypeStruct((B, S, D), W_pos.dtype),
    )(W_pos[:S])


# speedup vs baseline: 51.1515x; 1.6192x over previous
import jax
import jax.numpy as jnp
from jax.experimental import pallas as pl
from jax.experimental.pallas import tpu as pltpu


def kernel(tokens, W_pos):
    B = tokens.shape[0]
    S = tokens.shape[1]
    D = W_pos.shape[1]

    def body(w_hbm, out_hbm, vmem0, vmem1, in_sem, out_sem):
        c0 = pltpu.async_copy(w_hbm, vmem0, in_sem)
        c1 = pltpu.async_copy(w_hbm, vmem1, in_sem)
        c0.wait()
        c1.wait()
        srcs = [vmem0, vmem1]
        copies = [
            pltpu.async_copy(srcs[b % 2], out_hbm.at[b], out_sem)
            for b in range(B)
        ]
        for c in copies:
            c.wait()

    return pl.pallas_call(
        body,
        in_specs=[pl.BlockSpec(memory_space=pltpu.MemorySpace.HBM)],
        out_specs=pl.BlockSpec(memory_space=pltpu.MemorySpace.HBM),
        out_shape=jax.ShapeDtypeStruct((B, S, D), W_pos.dtype),
        scratch_shapes=[
            pltpu.VMEM((S, D), W_pos.dtype),
            pltpu.VMEM((S, D), W_pos.dtype),
            pltpu.SemaphoreType.DMA,
            pltpu.SemaphoreType.DMA,
        ],
    )(W_pos[:S])


# speedup vs baseline: 60.2835x; 1.1785x over previous
import jax
import jax.numpy as jnp
from jax.experimental import pallas as pl
from jax.experimental.pallas import tpu as pltpu

_CHUNKS = 1


def kernel(tokens, W_pos):
    B = tokens.shape[0]
    S = tokens.shape[1]
    D = W_pos.shape[1]
    rc = S // _CHUNKS

    def body(w_hbm, out_hbm, vmem, in_sem, out_sem):
        in_copies = [
            pltpu.make_async_copy(
                w_hbm.at[pl.ds(i * rc, rc)], vmem.at[pl.ds(i * rc, rc)], in_sem
            )
            for i in range(_CHUNKS)
        ]
        in_copies[0].start()
        out_copies = []
        for i in range(_CHUNKS):
            in_copies[i].wait()
            if i + 1 < _CHUNKS:
                in_copies[i + 1].start()
            for b in range(B):
                c = pltpu.async_copy(
                    vmem.at[pl.ds(i * rc, rc)],
                    out_hbm.at[b, pl.ds(i * rc, rc)],
                    out_sem,
                    priority=b % 2,
                )
                out_copies.append(c)
        for c in out_copies:
            c.wait()

    return pl.pallas_call(
        body,
        in_specs=[pl.BlockSpec(memory_space=pltpu.MemorySpace.HBM)],
        out_specs=pl.BlockSpec(memory_space=pltpu.MemorySpace.HBM),
        out_shape=jax.ShapeDtypeStruct((B, S, D), W_pos.dtype),
        scratch_shapes=[
            pltpu.VMEM((S, D), W_pos.dtype),
            pltpu.SemaphoreType.DMA,
            pltpu.SemaphoreType.DMA,
        ],
    )(W_pos[:S])


# speedup vs baseline: 60.4394x; 1.0026x over previous
import jax
import jax.numpy as jnp
from jax.experimental import pallas as pl
from jax.experimental.pallas import tpu as pltpu

_SPLIT = 2


def kernel(tokens, W_pos):
    B = tokens.shape[0]
    S = tokens.shape[1]
    D = W_pos.shape[1]
    rs = S // _SPLIT

    def body(w_hbm, out_hbm, vmem, in_sem, out_sem):
        pltpu.make_async_copy(w_hbm, vmem, in_sem).start()
        pltpu.make_async_copy(w_hbm, vmem, in_sem).wait()
        copies = [
            pltpu.async_copy(
                vmem.at[pl.ds(j * rs, rs)],
                out_hbm.at[b, pl.ds(j * rs, rs)],
                out_sem,
            )
            for j in range(_SPLIT)
            for b in range(B)
        ]
        for c in copies:
            c.wait()

    return pl.pallas_call(
        body,
        in_specs=[pl.BlockSpec(memory_space=pltpu.MemorySpace.HBM)],
        out_specs=pl.BlockSpec(memory_space=pltpu.MemorySpace.HBM),
        out_shape=jax.ShapeDtypeStruct((B, S, D), W_pos.dtype),
        scratch_shapes=[
            pltpu.VMEM((S, D), W_pos.dtype),
            pltpu.SemaphoreType.DMA,
            pltpu.SemaphoreType.DMA,
        ],
    )(W_pos[:S])
